# baseline (speedup 1.0000x reference)
import functools

import jax
import jax.numpy as jnp
from jax import lax
from jax.experimental import pallas as pl
from jax.experimental.pallas import tpu as pltpu
from jax.experimental.pallas import tpu_sc as plsc

N = 10000
E = 320000
IN_DIM = 128
HID = 64
NL = 3
EIN = 2 * HID + 1

NC = 2
NS = 16
LANES = 16
NW = NC * NS
CHUNK = 512
CPW = 20
EPW = CHUNK * CPW
E_PAD = EPW * NW
N_PAD = 10240
RPS = N_PAD // NS

_MESH = plsc.VectorSubcoreMesh(
    core_axis_name="c", subcore_axis_name="s", num_cores=NC, num_subcores=NS
)


def _wid():
    return lax.axis_index("s") * NC + lax.axis_index("c")


def _make_gather(W, dtype=jnp.float32):
    def body(tab_hbm, src_hbm, dst_hbm, out_hbm,
             idx0, idx1, rows0, rows1, si0, si1, sg0, sg1, sw0, sw1):
        base = _wid() * EPW
        idx_v = [idx0, idx1]
        rows_v = [rows0, rows1]
        s_i = [si0, si1]
        s_g = [sg0, sg1]
        s_w = [sw0, sw1]
        sides = [dst_hbm, src_hbm]
        T = 2 * CPW

        def off(t):
            return base + (t // 2) * CHUNK

        def col(t):
            return (t % 2) * W

        pend_w = [None, None]
        pend_i = [None, None]
        for b in range(2):
            pend_i[b] = pltpu.async_copy(
                sides[b % 2].at[pl.ds(off(b), CHUNK)], idx_v[b], s_i[b]
            )
        for t in range(T):
            b = t % 2
            pend_i[b].wait()
            if pend_w[b] is not None:
                pend_w[b].wait()
            pltpu.async_copy(tab_hbm.at[idx_v[b]], rows_v[b], s_g[b]).wait()
            pend_w[b] = pltpu.async_copy(
                rows_v[b],
                out_hbm.at[pl.ds(off(t), CHUNK), pl.ds(col(t), W)],
                s_w[b],
            )
            if t + 2 < T:
                pend_i[b] = pltpu.async_copy(
                    sides[t % 2].at[pl.ds(off(t + 2), CHUNK)], idx_v[b], s_i[b]
                )
        for b in range(2):
            pend_w[b].wait()

    return pl.kernel(
        body,
        out_type=jax.ShapeDtypeStruct((E_PAD, 2 * W), dtype),
        mesh=_MESH,
        scratch_types=[
            pltpu.VMEM((CHUNK,), jnp.int32),
            pltpu.VMEM((CHUNK,), jnp.int32),
            pltpu.VMEM((CHUNK, W), dtype),
            pltpu.VMEM((CHUNK, W), dtype),
            pltpu.SemaphoreType.DMA,
            pltpu.SemaphoreType.DMA,
            pltpu.SemaphoreType.DMA,
            pltpu.SemaphoreType.DMA,
            pltpu.SemaphoreType.DMA,
            pltpu.SemaphoreType.DMA,
        ],
        compiler_params=pltpu.CompilerParams(use_tc_tiling_on_sc=False),
    )


_gather = _make_gather(HID, jnp.bfloat16)
_cgather = _make_gather(16)


def _scatter_body(m_hbm, dst_hbm, out_hbm,
                  idx0, idx1, rows0, rows1, zv, acc_sh,
                  si0, si1, sm0, sm1, ss0, ss1):
    c = lax.axis_index("c")
    s = lax.axis_index("s")
    wid = s * NC + c
    base = wid * EPW
    idx_v = [idx0, idx1]
    rows_v = [rows0, rows1]
    s_i = [si0, si1]
    s_m = [sm0, sm1]
    s_s = [ss0, ss1]

    pend_i = [None, None]
    pend_m = [None, None]
    for b in range(2):
        off = base + b * CHUNK
        pend_i[b] = pltpu.async_copy(
            dst_hbm.at[pl.ds(off, CHUNK)], idx_v[b], s_i[b]
        )
        pend_m[b] = pltpu.async_copy(
            m_hbm.at[pl.ds(off, CHUNK)], rows_v[b], s_m[b]
        )

    ZR = 64
    for r in range(ZR):
        for q in range(HID // LANES):
            zv[r, pl.ds(q * LANES, LANES)] = jnp.zeros((LANES,), jnp.float32)
    def zrow(k, carry):
        pltpu.sync_copy(zv, acc_sh.at[pl.ds(s * RPS + k * ZR, ZR)])
        return carry
    lax.fori_loop(0, RPS // ZR, zrow, 0)
    plsc.subcore_barrier()

    for t in range(CPW):
        b = t % 2
        pend_i[b].wait()
        pend_m[b].wait()
        pltpu.async_copy(rows_v[b], acc_sh.at[idx_v[b]], s_s[b], add=True).wait()
        if t + 2 < CPW:
            off = base + (t + 2) * CHUNK
            pend_i[b] = pltpu.async_copy(
                dst_hbm.at[pl.ds(off, CHUNK)], idx_v[b], s_i[b]
            )
            pend_m[b] = pltpu.async_copy(
                m_hbm.at[pl.ds(off, CHUNK)], rows_v[b], s_m[b]
            )

    plsc.subcore_barrier()
    pltpu.sync_copy(
        acc_sh.at[pl.ds(s * RPS, RPS)], out_hbm.at[c, pl.ds(s * RPS, RPS)]
    )


_scatter = pl.kernel(
    _scatter_body,
    out_type=jax.ShapeDtypeStruct((NC, N_PAD, HID), jnp.float32),
    mesh=_MESH,
    scratch_types=[
        pltpu.VMEM((CHUNK,), jnp.int32),
        pltpu.VMEM((CHUNK,), jnp.int32),
        pltpu.VMEM((CHUNK, HID), jnp.float32),
        pltpu.VMEM((CHUNK, HID), jnp.float32),
        pltpu.VMEM((64, HID), jnp.float32),
        pltpu.VMEM_SHARED((N_PAD, HID), jnp.float32),
        pltpu.SemaphoreType.DMA,
        pltpu.SemaphoreType.DMA,
        pltpu.SemaphoreType.DMA,
        pltpu.SemaphoreType.DMA,
        pltpu.SemaphoreType.DMA,
        pltpu.SemaphoreType.DMA,
    ],
    compiler_params=pltpu.CompilerParams(use_tc_tiling_on_sc=False),
)


def _silu(t):
    return t * jax.nn.sigmoid(t)


def _h0_body(xh_ref, wt_ref, bt_ref, out_ref, outb_ref):
    t = jnp.dot(xh_ref[...], wt_ref[...], preferred_element_type=jnp.float32)
    t = t + bt_ref[...]
    h = jnp.where(t > 0.0, t, jnp.exp(t) - 1.0)
    out_ref[...] = h
    outb_ref[...] = h.astype(jnp.bfloat16)


def _h0(xh, W_t, b_t):
    return pl.pallas_call(
        _h0_body,
        out_shape=[
            jax.ShapeDtypeStruct((N, HID), jnp.float32),
            jax.ShapeDtypeStruct((N, HID), jnp.bfloat16),
        ],
    )(xh, W_t, b_t.reshape(1, HID))


def _edge_body(he_ref, ce_ref, w12_ref, wq_ref, b1_ref, w2_ref, b2_ref,
               out_ref):
    d = ce_ref[:, 0:16] - ce_ref[:, 16:32]
    t = jnp.dot(he_ref[...], w12_ref[...], preferred_element_type=jnp.float32)
    t = t + jnp.dot(d * d, wq_ref[...], preferred_element_type=jnp.float32)
    t = t + b1_ref[...]
    u = jnp.dot(
        _silu(t).astype(jnp.bfloat16),
        w2_ref[...].astype(jnp.bfloat16),
        preferred_element_type=jnp.float32,
    )
    out_ref[...] = _silu(u + b2_ref[...])


def _edge_mlp(he, ce, w12, wq, b1, w2, b2, blk):
    grid = (E_PAD // blk,)
    return pl.pallas_call(
        _edge_body,
        grid=grid,
        in_specs=[
            pl.BlockSpec((blk, 2 * HID), lambda i: (i, 0)),
            pl.BlockSpec((blk, 32), lambda i: (i, 0)),
            pl.BlockSpec((2 * HID, 2 * EIN), lambda i: (0, 0)),
            pl.BlockSpec((16, 2 * EIN), lambda i: (0, 0)),
            pl.BlockSpec((1, 2 * EIN), lambda i: (0, 0)),
            pl.BlockSpec((2 * EIN, HID), lambda i: (0, 0)),
            pl.BlockSpec((1, HID), lambda i: (0, 0)),
        ],
        out_specs=pl.BlockSpec((blk, HID), lambda i: (i, 0)),
        out_shape=jax.ShapeDtypeStruct((E_PAD, HID), jnp.float32),
    )(he, ce, w12, wq, b1, w2, b2)


def _node_body(h_ref, p_ref, u_ref, v_ref, b1_ref, w2_ref, b2_ref,
               out_ref, outb_ref):
    m = p_ref[0] + p_ref[1]
    t = (
        jnp.dot(h_ref[...], u_ref[...], preferred_element_type=jnp.float32)
        + jnp.dot(m, v_ref[...], preferred_element_type=jnp.float32)
        + b1_ref[...]
    )
    upd = jnp.dot(_silu(t), w2_ref[...], preferred_element_type=jnp.float32)
    h_new = h_ref[...] + 0.5 * (upd + b2_ref[...])
    out_ref[...] = h_new
    outb_ref[...] = h_new.astype(jnp.bfloat16)


def _node_mlp(h, parts, u, v, b1, w2, b2):
    return pl.pallas_call(
        _node_body,
        grid=(1,),
        in_specs=[
            pl.BlockSpec((N, HID), lambda i: (0, 0)),
            pl.BlockSpec((NC, N, HID), lambda i: (0, 0, 0)),
            pl.BlockSpec((HID, 2 * HID), lambda i: (0, 0)),
            pl.BlockSpec((HID, 2 * HID), lambda i: (0, 0)),
            pl.BlockSpec((1, 2 * HID), lambda i: (0, 0)),
            pl.BlockSpec((2 * HID, HID), lambda i: (0, 0)),
            pl.BlockSpec((1, HID), lambda i: (0, 0)),
        ],
        out_specs=[
            pl.BlockSpec((N, HID), lambda i: (0, 0)),
            pl.BlockSpec((N, HID), lambda i: (0, 0)),
        ],
        out_shape=[
            jax.ShapeDtypeStruct((N, HID), jnp.float32),
            jax.ShapeDtypeStruct((N, HID), jnp.bfloat16),
        ],
    )(h, parts, u, v, b1, w2, b2)


def _bn_body(h_ref, g_ref, b_ref, out_ref):
    h = h_ref[...]
    mean = jnp.mean(h, axis=0, keepdims=True)
    var = jnp.mean((h - mean) * (h - mean), axis=0, keepdims=True)
    out_ref[...] = (h - mean) * lax.rsqrt(var + 1e-5) * g_ref[...] + b_ref[...]


def _bn(h, gamma, beta):
    return pl.pallas_call(
        _bn_body,
        out_shape=jax.ShapeDtypeStruct((N, HID), jnp.float32),
    )(h, gamma.reshape(1, HID), beta.reshape(1, HID))


def kernel(x, edge_index, batch, W_t, b_t, ew1, eb1, ew2, eb2, nw1, nb1,
           nw2, nb2, cw1, cb1, cw2, cb2, gamma, beta):
    del batch, cw1, cb1, cw2, cb2
    ctab = jnp.pad(x[:, :3], ((0, 0), (0, 13)))
    xh = x[:, 3:]
    src = edge_index[0]
    dst = edge_index[1]
    pad = E_PAD - E
    zpad = jnp.zeros((pad,), jnp.int32)
    src_g = jnp.concatenate([src, zpad])
    dst_g = jnp.concatenate([dst, zpad])
    dst_s = jnp.concatenate([dst, jnp.full((pad,), N, jnp.int32)])

    h, hb = _h0(xh, W_t, b_t)
    ce = _cgather(ctab, src_g, dst_g)

    for l in range(NL):
        he = _gather(hb, src_g, dst_g)
        wq = jnp.ones((16, 1), jnp.float32) * ew1[l, 2 * HID].reshape(1, 2 * EIN)
        m = _edge_mlp(
            he, ce,
            ew1[l, : 2 * HID].astype(jnp.bfloat16), wq,
            eb1[l].reshape(1, 2 * EIN), ew2[l], eb2[l].reshape(1, HID),
            blk=1024,
        )
        parts = _scatter(m, dst_s)
        h, hb = _node_mlp(
            h, parts,
            nw1[l, :HID], nw1[l, HID:], nb1[l].reshape(1, 2 * HID),
            nw2[l], nb2[l].reshape(1, HID),
        )

    return _bn(h, gamma, beta)


# speedup vs baseline: 1.1634x; 1.1634x over previous
import functools

import jax
import jax.numpy as jnp
from jax import lax
from jax.experimental import pallas as pl
from jax.experimental.pallas import tpu as pltpu
from jax.experimental.pallas import tpu_sc as plsc

N = 10000
E = 320000
IN_DIM = 128
HID = 64
NL = 3
EIN = 2 * HID + 1

NC = 2
NS = 16
LANES = 16
NW = NC * NS
CHUNK = 512
CPW = 20
EPW = CHUNK * CPW
E_PAD = EPW * NW
N_PAD = 10240
RPS = N_PAD // NS

_MESH = plsc.VectorSubcoreMesh(
    core_axis_name="c", subcore_axis_name="s", num_cores=NC, num_subcores=NS
)


def _wid():
    return lax.axis_index("s") * NC + lax.axis_index("c")


def _make_gather(W, dtype=jnp.float32):
    def body(tab_hbm, src_hbm, dst_hbm, out_hbm,
             idx0, idx1, rows0, rows1, si0, si1, sg0, sg1, sw0, sw1):
        base = _wid() * EPW
        idx_v = [idx0, idx1]
        rows_v = [rows0, rows1]
        s_i = [si0, si1]
        s_g = [sg0, sg1]
        s_w = [sw0, sw1]
        sides = [dst_hbm, src_hbm]
        T = 2 * CPW

        def off(t):
            return base + (t // 2) * CHUNK

        def col(t):
            return (t % 2) * W

        pend_w = [None, None]
        pend_i = [None, None]
        for b in range(2):
            pend_i[b] = pltpu.async_copy(
                sides[b % 2].at[pl.ds(off(b), CHUNK)], idx_v[b], s_i[b]
            )
        for t in range(T):
            b = t % 2
            pend_i[b].wait()
            if pend_w[b] is not None:
                pend_w[b].wait()
            pltpu.async_copy(tab_hbm.at[idx_v[b]], rows_v[b], s_g[b]).wait()
            pend_w[b] = pltpu.async_copy(
                rows_v[b],
                out_hbm.at[pl.ds(off(t), CHUNK), pl.ds(col(t), W)],
                s_w[b],
            )
            if t + 2 < T:
                pend_i[b] = pltpu.async_copy(
                    sides[t % 2].at[pl.ds(off(t + 2), CHUNK)], idx_v[b], s_i[b]
                )
        for b in range(2):
            pend_w[b].wait()

    return pl.kernel(
        body,
        out_type=jax.ShapeDtypeStruct((E_PAD, 2 * W), dtype),
        mesh=_MESH,
        scratch_types=[
            pltpu.VMEM((CHUNK,), jnp.int32),
            pltpu.VMEM((CHUNK,), jnp.int32),
            pltpu.VMEM((CHUNK, W), dtype),
            pltpu.VMEM((CHUNK, W), dtype),
            pltpu.SemaphoreType.DMA,
            pltpu.SemaphoreType.DMA,
            pltpu.SemaphoreType.DMA,
            pltpu.SemaphoreType.DMA,
            pltpu.SemaphoreType.DMA,
            pltpu.SemaphoreType.DMA,
        ],
        compiler_params=pltpu.CompilerParams(use_tc_tiling_on_sc=False),
    )


_gather = _make_gather(HID)
_cgather = _make_gather(16)


def _scatter_body(m_hbm, dst_hbm, out_hbm,
                  idx0, idx1, rows0, rows1, zv, acc_sh,
                  si0, si1, sm0, sm1, ss0, ss1):
    c = lax.axis_index("c")
    s = lax.axis_index("s")
    wid = s * NC + c
    base = wid * EPW
    idx_v = [idx0, idx1]
    rows_v = [rows0, rows1]
    s_i = [si0, si1]
    s_m = [sm0, sm1]
    s_s = [ss0, ss1]

    pend_i = [None, None]
    pend_m = [None, None]
    for b in range(2):
        off = base + b * CHUNK
        pend_i[b] = pltpu.async_copy(
            dst_hbm.at[pl.ds(off, CHUNK)], idx_v[b], s_i[b]
        )
        pend_m[b] = pltpu.async_copy(
            m_hbm.at[pl.ds(off, CHUNK), pl.ds(0, HID)], rows_v[b], s_m[b]
        )

    ZR = 64
    for r in range(ZR):
        for q in range(HID // LANES):
            zv[r, pl.ds(q * LANES, LANES)] = jnp.zeros((LANES,), jnp.float32)
    def zrow(k, carry):
        pltpu.sync_copy(zv, acc_sh.at[pl.ds(s * RPS + k * ZR, ZR)])
        return carry
    lax.fori_loop(0, RPS // ZR, zrow, 0)
    plsc.subcore_barrier()

    for t in range(CPW):
        b = t % 2
        pend_i[b].wait()
        pend_m[b].wait()
        pltpu.async_copy(rows_v[b], acc_sh.at[idx_v[b]], s_s[b], add=True).wait()
        if t + 2 < CPW:
            off = base + (t + 2) * CHUNK
            pend_i[b] = pltpu.async_copy(
                dst_hbm.at[pl.ds(off, CHUNK)], idx_v[b], s_i[b]
            )
            pend_m[b] = pltpu.async_copy(
                m_hbm.at[pl.ds(off, CHUNK), pl.ds(0, HID)], rows_v[b], s_m[b]
            )

    plsc.subcore_barrier()
    pltpu.sync_copy(
        acc_sh.at[pl.ds(s * RPS, RPS)], out_hbm.at[c, pl.ds(s * RPS, RPS)]
    )


_scatter = pl.kernel(
    _scatter_body,
    out_type=jax.ShapeDtypeStruct((NC, N_PAD, HID), jnp.float32),
    mesh=_MESH,
    scratch_types=[
        pltpu.VMEM((CHUNK,), jnp.int32),
        pltpu.VMEM((CHUNK,), jnp.int32),
        pltpu.VMEM((CHUNK, HID), jnp.float32),
        pltpu.VMEM((CHUNK, HID), jnp.float32),
        pltpu.VMEM((64, HID), jnp.float32),
        pltpu.VMEM_SHARED((N_PAD, HID), jnp.float32),
        pltpu.SemaphoreType.DMA,
        pltpu.SemaphoreType.DMA,
        pltpu.SemaphoreType.DMA,
        pltpu.SemaphoreType.DMA,
        pltpu.SemaphoreType.DMA,
        pltpu.SemaphoreType.DMA,
    ],
    compiler_params=pltpu.CompilerParams(use_tc_tiling_on_sc=False),
)


def _silu(t):
    return t * jax.nn.sigmoid(t)


def _h0_body(xh_ref, wt_ref, bt_ref, out_ref):
    t = jnp.dot(xh_ref[...], wt_ref[...], preferred_element_type=jnp.float32)
    t = t + bt_ref[...]
    out_ref[...] = jnp.where(t > 0.0, t, jnp.exp(t) - 1.0)


def _h0(xh, W_t, b_t):
    return pl.pallas_call(
        _h0_body,
        out_shape=jax.ShapeDtypeStruct((N, HID), jnp.float32),
    )(xh, W_t, b_t.reshape(1, HID))


def _edge_body(he_ref, ce_ref, w12_ref, wq_ref, b1_ref, w2_ref, b2_ref,
               out_ref):
    d = ce_ref[:, 0:16] - ce_ref[:, 16:32]
    t = jnp.dot(he_ref[...], w12_ref[...], preferred_element_type=jnp.float32)
    t = t + jnp.dot(d * d, wq_ref[...], preferred_element_type=jnp.float32)
    t = t + b1_ref[...]
    u = jnp.dot(
        _silu(t).astype(jnp.bfloat16),
        w2_ref[...].astype(jnp.bfloat16),
        preferred_element_type=jnp.float32,
    )
    m = _silu(u + b2_ref[...])
    out_ref[...] = jnp.concatenate(
        [m, jnp.zeros_like(m)], axis=1
    )


def _edge_mlp(he, ce, w12, wq, b1, w2, b2, blk):
    grid = (E_PAD // blk,)
    return pl.pallas_call(
        _edge_body,
        grid=grid,
        in_specs=[
            pl.BlockSpec((blk, 2 * HID), lambda i: (i, 0)),
            pl.BlockSpec((blk, 32), lambda i: (i, 0)),
            pl.BlockSpec((2 * HID, 2 * EIN), lambda i: (0, 0)),
            pl.BlockSpec((16, 2 * EIN), lambda i: (0, 0)),
            pl.BlockSpec((1, 2 * EIN), lambda i: (0, 0)),
            pl.BlockSpec((2 * EIN, HID), lambda i: (0, 0)),
            pl.BlockSpec((1, HID), lambda i: (0, 0)),
        ],
        out_specs=pl.BlockSpec((blk, 2 * HID), lambda i: (i, 0)),
        out_shape=jax.ShapeDtypeStruct((E_PAD, 2 * HID), jnp.float32),
    )(he, ce, w12, wq, b1, w2, b2)


def _node_body(h_ref, p_ref, u_ref, v_ref, b1_ref, w2_ref, b2_ref, out_ref):
    m = p_ref[0] + p_ref[1]
    t = (
        jnp.dot(h_ref[...], u_ref[...], preferred_element_type=jnp.float32)
        + jnp.dot(m, v_ref[...], preferred_element_type=jnp.float32)
        + b1_ref[...]
    )
    upd = jnp.dot(_silu(t), w2_ref[...], preferred_element_type=jnp.float32)
    out_ref[...] = h_ref[...] + 0.5 * (upd + b2_ref[...])


def _node_mlp(h, parts, u, v, b1, w2, b2):
    return pl.pallas_call(
        _node_body,
        grid=(1,),
        in_specs=[
            pl.BlockSpec((N, HID), lambda i: (0, 0)),
            pl.BlockSpec((NC, N, HID), lambda i: (0, 0, 0)),
            pl.BlockSpec((HID, 2 * HID), lambda i: (0, 0)),
            pl.BlockSpec((HID, 2 * HID), lambda i: (0, 0)),
            pl.BlockSpec((1, 2 * HID), lambda i: (0, 0)),
            pl.BlockSpec((2 * HID, HID), lambda i: (0, 0)),
            pl.BlockSpec((1, HID), lambda i: (0, 0)),
        ],
        out_specs=pl.BlockSpec((N, HID), lambda i: (0, 0)),
        out_shape=jax.ShapeDtypeStruct((N, HID), jnp.float32),
    )(h, parts, u, v, b1, w2, b2)


def _bn_body(h_ref, g_ref, b_ref, out_ref):
    h = h_ref[...]
    mean = jnp.mean(h, axis=0, keepdims=True)
    var = jnp.mean((h - mean) * (h - mean), axis=0, keepdims=True)
    out_ref[...] = (h - mean) * lax.rsqrt(var + 1e-5) * g_ref[...] + b_ref[...]


def _bn(h, gamma, beta):
    return pl.pallas_call(
        _bn_body,
        out_shape=jax.ShapeDtypeStruct((N, HID), jnp.float32),
    )(h, gamma.reshape(1, HID), beta.reshape(1, HID))


def kernel(x, edge_index, batch, W_t, b_t, ew1, eb1, ew2, eb2, nw1, nb1,
           nw2, nb2, cw1, cb1, cw2, cb2, gamma, beta):
    del batch, cw1, cb1, cw2, cb2
    ctab = jnp.pad(x[:, :3], ((0, 0), (0, 13)))
    xh = x[:, 3:]
    src = edge_index[0]
    dst = edge_index[1]
    pad = E_PAD - E
    zpad = jnp.zeros((pad,), jnp.int32)
    src_g = jnp.concatenate([src, zpad])
    dst_g = jnp.concatenate([dst, zpad])
    dst_s = jnp.concatenate([dst, jnp.full((pad,), N, jnp.int32)])

    h = _h0(xh, W_t, b_t)
    ce = _cgather(ctab, src_g, dst_g)

    for l in range(NL):
        he = _gather(h, src_g, dst_g)
        wq = jnp.ones((16, 1), jnp.float32) * ew1[l, 2 * HID].reshape(1, 2 * EIN)
        m = _edge_mlp(
            he, ce,
            ew1[l, : 2 * HID].astype(jnp.bfloat16), wq,
            eb1[l].reshape(1, 2 * EIN), ew2[l], eb2[l].reshape(1, HID),
            blk=1024,
        )
        parts = _scatter(m, dst_s)
        h = _node_mlp(
            h, parts,
            nw1[l, :HID], nw1[l, HID:], nb1[l].reshape(1, 2 * HID),
            nw2[l], nb2[l].reshape(1, HID),
        )

    return _bn(h, gamma, beta)
